# baseline (speedup 1.0000x reference)
import functools

import jax
import jax.numpy as jnp
import numpy as np
from jax import lax
from jax.experimental import pallas as pl
from jax.experimental.pallas import tpu as pltpu
from jax.experimental.pallas import tpu_sc as plsc

NV, NF, E = 5000, 10000, 320000
DIM_CODEBOOK = 192
FACE_DIM = 208
ENC_DIMS = (64, 128, 256)

NC, NS = 2, 16
CHUNK = 96
KBUF = 4


def _round_up(x, m):
    return (x + m - 1) // m * m


@functools.partial(jax.jit, static_argnames=("m_pad", "w", "ngrp", "identity_src",
                                             "kbuf"))
def _sc_scatter_add(t0, t1, src4, dst4, zeros, *, m_pad, w, ngrp, identity_src,
                    kbuf=KBUF):
    mesh = plsc.VectorSubcoreMesh(
        core_axis_name="c", subcore_axis_name="s", num_cores=NC, num_subcores=NS)
    rpt = m_pad // NS

    def body(t0_ref, t1_ref, src_ref, dst_ref, zeros_ref, out0_ref, out1_ref,
             src_i, dst_i, rows_v, acc, gsems, ssems, isem):
        c = lax.axis_index("c")
        s = lax.axis_index("s")
        r0 = s * rpt
        if not identity_src:
            pltpu.async_copy(src_ref.at[s, 0], src_i.at[0], isem)
        pltpu.async_copy(dst_ref.at[s, 0], dst_i.at[0], isem)
        pltpu.sync_copy(zeros_ref.at[s], acc.at[pl.ds(r0, rpt)])
        plsc.subcore_barrier()

        def run(tref):
            def group(g, carry):
                par = lax.rem(g, 2)
                if not identity_src:
                    pltpu.make_async_copy(src_ref.at[s, g], src_i.at[par],
                                          isem).wait()
                pltpu.make_async_copy(dst_ref.at[s, g], dst_i.at[par],
                                      isem).wait()

                @pl.when(g + 1 < ngrp)
                def _():
                    if not identity_src:
                        pltpu.async_copy(src_ref.at[s, g + 1],
                                         src_i.at[1 - par], isem)
                    pltpu.async_copy(dst_ref.at[s, g + 1],
                                     dst_i.at[1 - par], isem)

                gathers = []
                for b in range(kbuf):
                    if identity_src:
                        row0 = ((s * ngrp + g) * kbuf + b) * CHUNK
                        srcslc = tref.at[pl.ds(row0, CHUNK)]
                    else:
                        srcslc = tref.at[src_i.at[par, b]]
                    gathers.append(pltpu.async_copy(srcslc, rows_v[b], gsems[b]))
                scatters = []
                for b in range(kbuf):
                    gathers[b].wait()
                    scatters.append(pltpu.async_copy(
                        rows_v[b], acc.at[dst_i.at[par, b]], ssems[b], add=True))
                for sd in scatters:
                    sd.wait()
                return carry

            lax.fori_loop(0, ngrp, group, 0)

        @pl.when(c == 0)
        def _():
            run(t0_ref)

        @pl.when(c == 1)
        def _():
            run(t1_ref)

        plsc.subcore_barrier()

        @pl.when(c == 0)
        def _():
            pltpu.sync_copy(acc.at[pl.ds(r0, rpt)], out0_ref.at[s])

        @pl.when(c == 1)
        def _():
            pltpu.sync_copy(acc.at[pl.ds(r0, rpt)], out1_ref.at[s])

    scratch = [
        pltpu.VMEM((2, kbuf, CHUNK), jnp.int32),
        pltpu.VMEM((2, kbuf, CHUNK), jnp.int32),
        [pltpu.VMEM((CHUNK, w), jnp.float32) for _ in range(kbuf)],
        pltpu.VMEM_SHARED((m_pad, w), jnp.float32),
        [pltpu.SemaphoreType.DMA for _ in range(kbuf)],
        [pltpu.SemaphoreType.DMA for _ in range(kbuf)],
        pltpu.SemaphoreType.DMA,
    ]
    out_type = (jax.ShapeDtypeStruct((NS, rpt, w), jnp.float32),
                jax.ShapeDtypeStruct((NS, rpt, w), jnp.float32))
    o0, o1 = pl.kernel(
        body, out_type=out_type, mesh=mesh, scratch_types=scratch,
        compiler_params=pltpu.CompilerParams(use_tc_tiling_on_sc=False))(
        t0, t1, src4, dst4, zeros.reshape(NS, rpt, w))
    return o0.reshape(m_pad, w), o1.reshape(m_pad, w)


def _tc_call(body, n_rows, bm, in_kinds, out_ws, *args):
    grid = (n_rows // bm,)
    in_specs = []
    for a, kind in zip(args, in_kinds):
        if kind == "row":
            in_specs.append(pl.BlockSpec((bm, a.shape[1]), lambda i: (i, 0)))
        else:
            in_specs.append(pl.BlockSpec(a.shape, lambda i: (0,) * a.ndim))
    multi = isinstance(out_ws, (tuple, list))
    ws = out_ws if multi else [out_ws]
    out = pl.pallas_call(
        body,
        grid=grid,
        in_specs=in_specs,
        out_specs=[pl.BlockSpec((bm, w_), lambda i: (i, 0)) for w_ in ws],
        out_shape=[jax.ShapeDtypeStruct((n_rows, w_), jnp.float32) for w_ in ws],
    )(*args)
    return out if multi else out[0]


def _write_row_layout(ref, lo, w, bm, segs):
    for a, bnd, val in segs:
        aa, bb = max(a, lo), min(bnd, lo + w)
        if bb > aa:
            if isinstance(val, float):
                ref[:, aa - lo:bb - lo] = jnp.full((bm, bb - aa), val, jnp.float32)
            else:
                ref[:, aa - lo:bb - lo] = val[:, aa - a:bb - a]


def _aug_proj(x, Wp, bp, D, w, has_ones):
    bm = 400
    ones0 = max(D, w)

    def body(x_ref, w_ref, b_ref, o0_ref, o1_ref):
        xp = jnp.maximum(
            jnp.dot(x_ref[...], w_ref[...], preferred_element_type=jnp.float32)
            + b_ref[...], 0.0)
        segs = [(0, D, xp)]
        if has_ones:
            segs += [(D, ones0, 0.0), (ones0, ones0 + 16, 1.0),
                     (ones0 + 16, 2 * w, 0.0)]
        _write_row_layout(o0_ref, 0, w, bm, segs)
        _write_row_layout(o1_ref, w, w, bm, segs)

    return _tc_call(body, NF, bm, ("row", "full", "full"), (w, w), x, Wp, bp)


def _aug_proj_l1(x, Wp, bp, Wl, Wr, D, Dg, w):
    bm = 400
    ones0 = max(Dg, w)

    def body(x_ref, wp_ref, b_ref, wl_ref, wr_ref, o0_ref, o1_ref, xw_ref):
        xp = jnp.maximum(
            jnp.dot(x_ref[...], wp_ref[...], preferred_element_type=jnp.float32)
            + b_ref[...], 0.0)
        yp = jnp.dot(xp, wl_ref[...], preferred_element_type=jnp.float32)
        xw_ref[...] = jnp.dot(xp, wr_ref[...],
                              preferred_element_type=jnp.float32)
        segs = [(0, Dg, yp), (Dg, ones0, 0.0), (ones0, ones0 + 16, 1.0),
                (ones0 + 16, 2 * w, 0.0)]
        _write_row_layout(o0_ref, 0, w, bm, segs)
        _write_row_layout(o1_ref, w, w, bm, segs)

    return _tc_call(body, NF, bm, ("row", "full", "full", "full", "full"),
                    (w, w, Dg), x, Wp, bp, Wl, Wr)


def _sage_out_pre(m0, m1, xw, cnt16, b, Dg, w):
    bm = 400

    def body(m0_ref, m1_ref, xw_ref, cnt_ref, b_ref, o_ref):
        msg = jnp.concatenate([m0_ref[...], m1_ref[:, :Dg - w]], axis=1)
        cnt = cnt_ref[:, 0:1]
        out = msg / jnp.maximum(cnt, 1.0) + xw_ref[...] + b_ref[...]
        nrm = jnp.sqrt(jnp.sum(out * out, axis=1, keepdims=True))
        out = out / jnp.maximum(nrm, 1e-12)
        o_ref[...] = out * jax.nn.sigmoid(out)

    return _tc_call(body, NF, bm, ("row", "row", "row", "row", "full"),
                    Dg, m0, m1, xw, cnt16, b)


def _sage_out(m0, m1, t0, t1, cnt16, Wl, Wr, b, D, w, dout):
    bm = 400

    def body(m0_ref, m1_ref, t0_ref, t1_ref, cnt_ref, wl_ref, wr_ref, b_ref,
             o_ref):
        m0v = m0_ref[...]
        m1v = m1_ref[...]
        if D > w:
            msg = jnp.concatenate([m0v, m1v[:, :D - w]], axis=1)
            xp = jnp.concatenate([t0_ref[...], t1_ref[:, :D - w]], axis=1)
        else:
            msg = m0v[:, :D]
            xp = t0_ref[:, :D]
        cnt = cnt_ref[:, 0:1]
        mean = msg / jnp.maximum(cnt, 1.0)
        out = (jnp.dot(mean, wl_ref[...], preferred_element_type=jnp.float32)
               + jnp.dot(xp, wr_ref[...], preferred_element_type=jnp.float32)
               + b_ref[...])
        nrm = jnp.sqrt(jnp.sum(out * out, axis=1, keepdims=True))
        out = out / jnp.maximum(nrm, 1e-12)
        o_ref[...] = out * jax.nn.sigmoid(out)

    return _tc_call(body, NF, bm,
                    ("row", "row", "row", "row", "row", "full", "full", "full"),
                    dout, m0, m1, t0, t1, cnt16, Wl, Wr, b)


def _codes_aug(h, Wc, bc):
    bm = 400

    def body(h_ref, w_ref, b_ref, o0_ref, o1_ref):
        codes = (jnp.dot(h_ref[...], w_ref[...], preferred_element_type=jnp.float32)
                 + b_ref[...])
        for k in range(3):
            ck = codes[:, k * 192:(k + 1) * 192]
            segs = [(0, 192, ck), (192, 208, 1.0), (208, 224, 0.0)]
            for ref, lo in ((o0_ref, 0), (o1_ref, 112)):
                for a, bnd, val in segs:
                    aa, bb = max(a, lo), min(bnd, lo + 112)
                    if bb > aa:
                        if isinstance(val, float):
                            ref[:, k * 112 + aa - lo:k * 112 + bb - lo] = (
                                jnp.full((bm, bb - aa), val, jnp.float32))
                        else:
                            ref[:, k * 112 + aa - lo:k * 112 + bb - lo] = (
                                val[:, aa - a:bb - a])

    return _tc_call(body, NF, bm, ("row", "full", "full"), (336, 336), h, Wc, bc)


def _vert_div(m0, m1):
    bm = 200

    def body(m0_ref, m1_ref, o_ref):
        den = jnp.maximum(m1_ref[:, 80:81], 1e-5)
        o_ref[:, :112] = m0_ref[...] / den
        o_ref[:, 112:] = m1_ref[:, :80] / den

    return _tc_call(body, NV, bm, ("row", "row"), DIM_CODEBOOK, m0, m1)


def _l2norm(t, eps=1e-12):
    return t / jnp.clip(jnp.linalg.norm(t, axis=-1, keepdims=True), eps)


def _discretize(t, lo, hi, num=128):
    t = (t - lo) / (hi - lo) * num - 0.5
    return jnp.clip(jnp.round(t).astype(jnp.int32), 0, num - 1)


def _face_feats(vertices, faces, params):
    v = vertices[0]
    f = faces[0]
    fc = v[f]
    fc3 = jnp.pad(fc, ((0, 0), (0, 0), (0, 1)))
    shifted = jnp.concatenate([fc3[:, -1:], fc3[:, :-1]], axis=1)
    z = jnp.sum(_l2norm(fc3) * _l2norm(shifted), axis=-1)
    angles = jnp.arccos(jnp.clip(z, -1 + 1e-5, 1 - 1e-5))
    ev = fc3 - shifted
    cross = jnp.cross(ev[:, 0], ev[:, 1])
    normals = _l2norm(cross)
    area = jnp.linalg.norm(cross, axis=-1, keepdims=True) * 0.5
    ce = params['coor_embed'][_discretize(fc, -1.0, 1.0)].reshape(NF, -1)
    ae = params['angle_embed'][_discretize(angles, 0.0, float(np.pi))].reshape(NF, -1)
    re = params['area_embed'][_discretize(area, 0.0, 4.0)].reshape(NF, -1)
    ne = params['normal_embed'][_discretize(normals, -1.0, 1.0)].reshape(NF, -1)
    return jnp.concatenate([ce, ae, re, ne], axis=-1)


def kernel(vertices, faces, face_edges, params):
    x = _face_feats(vertices, faces, params)

    def edge_arrays(kbuf):
        egrain = NS * kbuf * CHUNK
        e_pad = _round_up(E, egrain)
        ngrp = e_pad // egrain
        s_ = jnp.pad(face_edges[0, :, 0], (0, e_pad - E)
                     ).reshape(NS, ngrp, kbuf, CHUNK)
        d_ = jnp.pad(face_edges[0, :, 1], (0, e_pad - E),
                     constant_values=NF).reshape(NS, ngrp, kbuf, CHUNK)
        return s_, d_, ngrp

    ecache = {}
    m_pad_f = _round_up(NF + 8, NS * 8)

    dims = [FACE_DIM] + list(ENC_DIMS)
    layer_kbuf = (8, 8, 8)
    h = x
    cnt16 = None
    for i, p in enumerate(params['sage']):
        D = dims[i]
        kbuf = layer_kbuf[i]
        if kbuf not in ecache:
            ecache[kbuf] = edge_arrays(kbuf)
        src, dst, ngrp_e = ecache[kbuf]
        if i == 0:
            Dg = dims[1]
            w = _round_up(Dg + 16, 32) // 2
            t0, t1, xw = _aug_proj_l1(h, p['Wp'], p['bp'].reshape(1, -1),
                                      p['Wl'], p['Wr'], D, Dg, w)
            zeros = jnp.zeros((m_pad_f, w), jnp.float32)
            m0, m1 = _sc_scatter_add(
                t0, t1, src, dst, zeros,
                m_pad=m_pad_f, w=w, ngrp=ngrp_e, identity_src=False, kbuf=kbuf)
            cnt0 = max(Dg, w) - w
            cnt16 = m1[:NF, cnt0:cnt0 + 16]
            h = _sage_out_pre(m0[:NF], m1[:NF], xw, cnt16,
                              p['b'].reshape(1, -1), Dg, w)
            continue
        w = D // 2
        t0, t1 = _aug_proj(h, p['Wp'], p['bp'].reshape(1, -1), D, w, False)
        zeros = jnp.zeros((m_pad_f, w), jnp.float32)
        m0, m1 = _sc_scatter_add(
            t0, t1, src, dst, zeros,
            m_pad=m_pad_f, w=w, ngrp=ngrp_e, identity_src=False, kbuf=kbuf)
        h = _sage_out(m0[:NF], m1[:NF], t0, t1, cnt16, p['Wl'], p['Wr'],
                      p['b'].reshape(1, -1), D, w, dims[i + 1])

    c0, c1 = _codes_aug(h, params['Wc'], params['bc'].reshape(1, -1))
    kb_v = 2
    vgrain = NS * kb_v * CHUNK
    ep = _round_up(3 * NF, vgrain)
    ngrp_v = ep // vgrain
    c0 = jnp.pad(c0.reshape(3 * NF, 112), ((0, ep - 3 * NF), (0, 0)))
    c1 = jnp.pad(c1.reshape(3 * NF, 112), ((0, ep - 3 * NF), (0, 0)))
    fdst = jnp.pad(faces[0].reshape(-1), (0, ep - 3 * NF),
                   constant_values=NV).reshape(NS, ngrp_v, kb_v, CHUNK)
    m_pad_v = _round_up(NV + 8, NS * 8)
    zeros_v = jnp.zeros((m_pad_v, 112), jnp.float32)
    vm0, vm1 = _sc_scatter_add(
        c0, c1, fdst, fdst, zeros_v,
        m_pad=m_pad_v, w=112, ngrp=ngrp_v, identity_src=True, kbuf=kb_v)
    out = _vert_div(vm0[:NV], vm1[:NV])
    return out.reshape(1, NV, DIM_CODEBOOK)


# speedup vs baseline: 1.1804x; 1.1804x over previous
import functools

import jax
import jax.numpy as jnp
import numpy as np
from jax import lax
from jax.experimental import pallas as pl
from jax.experimental.pallas import tpu as pltpu
from jax.experimental.pallas import tpu_sc as plsc

NV, NF, E = 5000, 10000, 320000
DIM_CODEBOOK = 192
FACE_DIM = 208
ENC_DIMS = (64, 128, 256)

NC, NS = 2, 16
CHUNK = 96
KBUF = 4


def _round_up(x, m):
    return (x + m - 1) // m * m


@functools.partial(jax.jit, static_argnames=("m_pad", "w", "ngrp", "identity_src",
                                             "kbuf"))
def _sc_scatter_add(t0, t1, src4, dst4, zeros, *, m_pad, w, ngrp, identity_src,
                    kbuf=KBUF):
    mesh = plsc.VectorSubcoreMesh(
        core_axis_name="c", subcore_axis_name="s", num_cores=NC, num_subcores=NS)
    rpt = m_pad // NS

    def body(t0_ref, t1_ref, src_ref, dst_ref, zeros_ref, out0_ref, out1_ref,
             src_i, dst_i, rows_v, acc, gsems, ssems, isem):
        c = lax.axis_index("c")
        s = lax.axis_index("s")
        r0 = s * rpt
        if not identity_src:
            pltpu.async_copy(src_ref.at[s, 0], src_i.at[0], isem)
        pltpu.async_copy(dst_ref.at[s, 0], dst_i.at[0], isem)
        pltpu.sync_copy(zeros_ref.at[s], acc.at[pl.ds(r0, rpt)])
        plsc.subcore_barrier()

        def run(tref):
            def group(g, carry):
                par = lax.rem(g, 2)
                if not identity_src:
                    pltpu.make_async_copy(src_ref.at[s, g], src_i.at[par],
                                          isem).wait()
                pltpu.make_async_copy(dst_ref.at[s, g], dst_i.at[par],
                                      isem).wait()

                @pl.when(g + 1 < ngrp)
                def _():
                    if not identity_src:
                        pltpu.async_copy(src_ref.at[s, g + 1],
                                         src_i.at[1 - par], isem)
                    pltpu.async_copy(dst_ref.at[s, g + 1],
                                     dst_i.at[1 - par], isem)

                gathers = []
                for b in range(kbuf):
                    if identity_src:
                        row0 = ((s * ngrp + g) * kbuf + b) * CHUNK
                        srcslc = tref.at[pl.ds(row0, CHUNK)]
                    else:
                        srcslc = tref.at[src_i.at[par, b]]
                    gathers.append(pltpu.async_copy(srcslc, rows_v[b], gsems[b]))
                scatters = []
                for b in range(kbuf):
                    gathers[b].wait()
                    scatters.append(pltpu.async_copy(
                        rows_v[b], acc.at[dst_i.at[par, b]], ssems[b], add=True))
                for sd in scatters:
                    sd.wait()
                return carry

            lax.fori_loop(0, ngrp, group, 0)

        @pl.when(c == 0)
        def _():
            run(t0_ref)

        @pl.when(c == 1)
        def _():
            run(t1_ref)

        plsc.subcore_barrier()

        @pl.when(c == 0)
        def _():
            pltpu.sync_copy(acc.at[pl.ds(r0, rpt)], out0_ref.at[s])

        @pl.when(c == 1)
        def _():
            pltpu.sync_copy(acc.at[pl.ds(r0, rpt)], out1_ref.at[s])

    scratch = [
        pltpu.VMEM((2, kbuf, CHUNK), jnp.int32),
        pltpu.VMEM((2, kbuf, CHUNK), jnp.int32),
        [pltpu.VMEM((CHUNK, w), jnp.float32) for _ in range(kbuf)],
        pltpu.VMEM_SHARED((m_pad, w), jnp.float32),
        [pltpu.SemaphoreType.DMA for _ in range(kbuf)],
        [pltpu.SemaphoreType.DMA for _ in range(kbuf)],
        pltpu.SemaphoreType.DMA,
    ]
    out_type = (jax.ShapeDtypeStruct((NS, rpt, w), jnp.float32),
                jax.ShapeDtypeStruct((NS, rpt, w), jnp.float32))
    o0, o1 = pl.kernel(
        body, out_type=out_type, mesh=mesh, scratch_types=scratch,
        compiler_params=pltpu.CompilerParams(use_tc_tiling_on_sc=False))(
        t0, t1, src4, dst4, zeros.reshape(NS, rpt, w))
    return o0.reshape(m_pad, w), o1.reshape(m_pad, w)


def _tc_call(body, n_rows, bm, in_kinds, out_ws, *args):
    grid = (n_rows // bm,)
    in_specs = []
    for a, kind in zip(args, in_kinds):
        if kind == "row":
            in_specs.append(pl.BlockSpec((bm, a.shape[1]), lambda i: (i, 0)))
        else:
            in_specs.append(pl.BlockSpec(a.shape, lambda i: (0,) * a.ndim))
    multi = isinstance(out_ws, (tuple, list))
    ws = out_ws if multi else [out_ws]
    out = pl.pallas_call(
        body,
        grid=grid,
        in_specs=in_specs,
        out_specs=[pl.BlockSpec((bm, w_), lambda i: (i, 0)) for w_ in ws],
        out_shape=[jax.ShapeDtypeStruct((n_rows, w_), jnp.float32) for w_ in ws],
    )(*args)
    return out if multi else out[0]


def _write_row_layout(ref, lo, w, bm, segs):
    for a, bnd, val in segs:
        aa, bb = max(a, lo), min(bnd, lo + w)
        if bb > aa:
            if isinstance(val, float):
                ref[:, aa - lo:bb - lo] = jnp.full((bm, bb - aa), val, jnp.float32)
            else:
                ref[:, aa - lo:bb - lo] = val[:, aa - a:bb - a]


def _aug_proj(x, Wp, bp, D, w, has_ones):
    bm = 400
    ones0 = max(D, w)

    def body(x_ref, w_ref, b_ref, o0_ref, o1_ref):
        xp = jnp.maximum(
            jnp.dot(x_ref[...], w_ref[...], preferred_element_type=jnp.float32)
            + b_ref[...], 0.0)
        segs = [(0, D, xp)]
        if has_ones:
            segs += [(D, ones0, 0.0), (ones0, ones0 + 16, 1.0),
                     (ones0 + 16, 2 * w, 0.0)]
        _write_row_layout(o0_ref, 0, w, bm, segs)
        _write_row_layout(o1_ref, w, w, bm, segs)

    return _tc_call(body, NF, bm, ("row", "full", "full"), (w, w), x, Wp, bp)


def _aug_proj_l1(x, Wp, bp, Wl, Wr, D, Dg, w):
    bm = 400
    ones0 = max(Dg, w)

    def body(x_ref, wp_ref, b_ref, wl_ref, wr_ref, o0_ref, o1_ref, xw_ref):
        xp = jnp.maximum(
            jnp.dot(x_ref[...], wp_ref[...], preferred_element_type=jnp.float32)
            + b_ref[...], 0.0)
        yp = jnp.dot(xp, wl_ref[...], preferred_element_type=jnp.float32)
        xw_ref[...] = jnp.dot(xp, wr_ref[...],
                              preferred_element_type=jnp.float32)
        segs = [(0, Dg, yp), (Dg, ones0, 0.0), (ones0, ones0 + 16, 1.0),
                (ones0 + 16, 2 * w, 0.0)]
        _write_row_layout(o0_ref, 0, w, bm, segs)
        _write_row_layout(o1_ref, w, w, bm, segs)

    return _tc_call(body, NF, bm, ("row", "full", "full", "full", "full"),
                    (w, w, Dg), x, Wp, bp, Wl, Wr)


def _sage_out_pre(m0, m1, xw, cnt16, b, Dg, w):
    bm = 400

    def body(m0_ref, m1_ref, xw_ref, cnt_ref, b_ref, o_ref):
        msg = jnp.concatenate([m0_ref[...], m1_ref[:, :Dg - w]], axis=1)
        cnt = cnt_ref[:, 0:1]
        out = msg / jnp.maximum(cnt, 1.0) + xw_ref[...] + b_ref[...]
        nrm = jnp.sqrt(jnp.sum(out * out, axis=1, keepdims=True))
        out = out / jnp.maximum(nrm, 1e-12)
        o_ref[...] = out * jax.nn.sigmoid(out)

    return _tc_call(body, NF, bm, ("row", "row", "row", "row", "full"),
                    Dg, m0, m1, xw, cnt16, b)


def _sage_out(m0, m1, t0, t1, cnt16, Wl, Wr, b, D, w, dout):
    bm = 400

    def body(m0_ref, m1_ref, t0_ref, t1_ref, cnt_ref, wl_ref, wr_ref, b_ref,
             o_ref):
        m0v = m0_ref[...]
        m1v = m1_ref[...]
        if D > w:
            msg = jnp.concatenate([m0v, m1v[:, :D - w]], axis=1)
            xp = jnp.concatenate([t0_ref[...], t1_ref[:, :D - w]], axis=1)
        else:
            msg = m0v[:, :D]
            xp = t0_ref[:, :D]
        cnt = cnt_ref[:, 0:1]
        mean = msg / jnp.maximum(cnt, 1.0)
        out = (jnp.dot(mean, wl_ref[...], preferred_element_type=jnp.float32)
               + jnp.dot(xp, wr_ref[...], preferred_element_type=jnp.float32)
               + b_ref[...])
        nrm = jnp.sqrt(jnp.sum(out * out, axis=1, keepdims=True))
        out = out / jnp.maximum(nrm, 1e-12)
        o_ref[...] = out * jax.nn.sigmoid(out)

    return _tc_call(body, NF, bm,
                    ("row", "row", "row", "row", "row", "full", "full", "full"),
                    dout, m0, m1, t0, t1, cnt16, Wl, Wr, b)


def _codes_aug(h, Wc, bc):
    bm = 400

    def body(h_ref, w_ref, b_ref, o0_ref, o1_ref):
        codes = (jnp.dot(h_ref[...], w_ref[...], preferred_element_type=jnp.float32)
                 + b_ref[...])
        for k in range(3):
            ck = codes[:, k * 192:(k + 1) * 192]
            segs = [(0, 192, ck), (192, 208, 1.0), (208, 224, 0.0)]
            for ref, lo in ((o0_ref, 0), (o1_ref, 112)):
                for a, bnd, val in segs:
                    aa, bb = max(a, lo), min(bnd, lo + 112)
                    if bb > aa:
                        if isinstance(val, float):
                            ref[:, k * 112 + aa - lo:k * 112 + bb - lo] = (
                                jnp.full((bm, bb - aa), val, jnp.float32))
                        else:
                            ref[:, k * 112 + aa - lo:k * 112 + bb - lo] = (
                                val[:, aa - a:bb - a])

    return _tc_call(body, NF, bm, ("row", "full", "full"), (336, 336), h, Wc, bc)


def _vert_div(m0, m1):
    bm = 200

    def body(m0_ref, m1_ref, o_ref):
        den = jnp.maximum(m1_ref[:, 80:81], 1e-5)
        o_ref[:, :112] = m0_ref[...] / den
        o_ref[:, 112:] = m1_ref[:, :80] / den

    return _tc_call(body, NV, bm, ("row", "row"), DIM_CODEBOOK, m0, m1)


def _l2norm(t, eps=1e-12):
    return t / jnp.clip(jnp.linalg.norm(t, axis=-1, keepdims=True), eps)


def _discretize(t, lo, hi, num=128):
    t = (t - lo) / (hi - lo) * num - 0.5
    return jnp.clip(jnp.round(t).astype(jnp.int32), 0, num - 1)


def _face_feats(vertices, faces, params):
    v = vertices[0]
    f = faces[0]
    fc = v[f]
    fc3 = jnp.pad(fc, ((0, 0), (0, 0), (0, 1)))
    shifted = jnp.concatenate([fc3[:, -1:], fc3[:, :-1]], axis=1)
    z = jnp.sum(_l2norm(fc3) * _l2norm(shifted), axis=-1)
    angles = jnp.arccos(jnp.clip(z, -1 + 1e-5, 1 - 1e-5))
    ev = fc3 - shifted
    cross = jnp.cross(ev[:, 0], ev[:, 1])
    normals = _l2norm(cross)
    area = jnp.linalg.norm(cross, axis=-1, keepdims=True) * 0.5
    ce = params['coor_embed'][_discretize(fc, -1.0, 1.0)].reshape(NF, -1)
    ae = params['angle_embed'][_discretize(angles, 0.0, float(np.pi))].reshape(NF, -1)
    re = params['area_embed'][_discretize(area, 0.0, 4.0)].reshape(NF, -1)
    ne = params['normal_embed'][_discretize(normals, -1.0, 1.0)].reshape(NF, -1)
    return jnp.concatenate([ce, ae, re, ne], axis=-1)


def kernel(vertices, faces, face_edges, params):
    x = _face_feats(vertices, faces, params)

    def edge_arrays(kbuf):
        egrain = NS * kbuf * CHUNK
        e_pad = _round_up(E, egrain)
        ngrp = e_pad // egrain
        s_ = jnp.pad(face_edges[0, :, 0], (0, e_pad - E)
                     ).reshape(NS, ngrp, kbuf, CHUNK)
        d_ = jnp.pad(face_edges[0, :, 1], (0, e_pad - E),
                     constant_values=NF).reshape(NS, ngrp, kbuf, CHUNK)
        return s_, d_, ngrp

    ecache = {}
    m_pad_f = _round_up(NF + 8, NS * 8)

    dims = [FACE_DIM] + list(ENC_DIMS)
    layer_kbuf = (4, 4, 4)
    h = x
    cnt16 = None
    for i, p in enumerate(params['sage']):
        D = dims[i]
        kbuf = layer_kbuf[i]
        if kbuf not in ecache:
            ecache[kbuf] = edge_arrays(kbuf)
        src, dst, ngrp_e = ecache[kbuf]
        if i == 0:
            Dg = dims[1]
            w = _round_up(Dg + 16, 32) // 2
            t0, t1, xw = _aug_proj_l1(h, p['Wp'], p['bp'].reshape(1, -1),
                                      p['Wl'], p['Wr'], D, Dg, w)
            zeros = jnp.zeros((m_pad_f, w), jnp.float32)
            m0, m1 = _sc_scatter_add(
                t0, t1, src, dst, zeros,
                m_pad=m_pad_f, w=w, ngrp=ngrp_e, identity_src=False, kbuf=kbuf)
            cnt0 = max(Dg, w) - w
            cnt16 = m1[:NF, cnt0:cnt0 + 16]
            h = _sage_out_pre(m0[:NF], m1[:NF], xw, cnt16,
                              p['b'].reshape(1, -1), Dg, w)
            continue
        w = D // 2
        t0, t1 = _aug_proj(h, p['Wp'], p['bp'].reshape(1, -1), D, w, False)
        zeros = jnp.zeros((m_pad_f, w), jnp.float32)
        m0, m1 = _sc_scatter_add(
            t0, t1, src, dst, zeros,
            m_pad=m_pad_f, w=w, ngrp=ngrp_e, identity_src=False, kbuf=kbuf)
        h = _sage_out(m0[:NF], m1[:NF], t0, t1, cnt16, p['Wl'], p['Wr'],
                      p['b'].reshape(1, -1), D, w, dims[i + 1])

    c0, c1 = _codes_aug(h, params['Wc'], params['bc'].reshape(1, -1))
    kb_v = 2
    vgrain = NS * kb_v * CHUNK
    ep = _round_up(3 * NF, vgrain)
    ngrp_v = ep // vgrain
    c0 = jnp.pad(c0.reshape(3 * NF, 112), ((0, ep - 3 * NF), (0, 0)))
    c1 = jnp.pad(c1.reshape(3 * NF, 112), ((0, ep - 3 * NF), (0, 0)))
    fdst = jnp.pad(faces[0].reshape(-1), (0, ep - 3 * NF),
                   constant_values=NV).reshape(NS, ngrp_v, kb_v, CHUNK)
    m_pad_v = _round_up(NV + 8, NS * 8)
    zeros_v = jnp.zeros((m_pad_v, 112), jnp.float32)
    vm0, vm1 = _sc_scatter_add(
        c0, c1, fdst, fdst, zeros_v,
        m_pad=m_pad_v, w=112, ngrp=ngrp_v, identity_src=True, kbuf=kb_v)
    out = _vert_div(vm0[:NV], vm1[:NV])
    return out.reshape(1, NV, DIM_CODEBOOK)
